# Initial kernel scaffold; baseline (speedup 1.0000x reference)
import jax
import jax.numpy as jnp
from jax.experimental import pallas as pl


def kernel(x, genres, collection, ov_emb, movie_table, emb_genre, emb_collection, W1, b1, W2, b2):
    raise NotImplementedError("write your pallas kernel here")


# speedup vs baseline: 1.0123x; 1.0123x over previous
import functools

import jax
import jax.numpy as jnp
from jax import lax
from jax.experimental import pallas as pl
from jax.experimental.pallas import tpu as pltpu
from jax.experimental.pallas import tpu_sc as plsc

B = 4096
D_MOVIE = 64
D_OV = 384
G = 4
GENRE_VOCAB = 32
DG = 32
DC = 32
HIDDEN = 64
RANK = 64

NC = 2
NS = 16
NW = NC * NS
BPW = B // NW


def _sc_gather(x, g_cols, coll, movie_table, ov_emb, emb_collection):
  mesh = plsc.VectorSubcoreMesh(
      core_axis_name="c", subcore_axis_name="s", num_cores=NC, num_subcores=NS)

  @functools.partial(
      pl.kernel,
      out_type=(
          jax.ShapeDtypeStruct((B, D_MOVIE), jnp.float32),
          jax.ShapeDtypeStruct((B, DC), jnp.float32),
          jax.ShapeDtypeStruct((B, D_OV), jnp.float32),
          jax.ShapeDtypeStruct((B,), jnp.int32),
          jax.ShapeDtypeStruct((B,), jnp.int32),
          jax.ShapeDtypeStruct((B,), jnp.int32),
          jax.ShapeDtypeStruct((B,), jnp.int32),
      ),
      mesh=mesh,
      compiler_params=pltpu.CompilerParams(use_tc_tiling_on_sc=False),
      scratch_types=[
          pltpu.VMEM((BPW,), jnp.int32),
          pltpu.VMEM((BPW,), jnp.int32),
          pltpu.VMEM((BPW,), jnp.int32),
          pltpu.VMEM((BPW,), jnp.int32),
          pltpu.VMEM((BPW,), jnp.int32),
          pltpu.VMEM((BPW,), jnp.int32),
          pltpu.VMEM((BPW, D_MOVIE), jnp.float32),
          pltpu.VMEM((BPW, D_OV), jnp.float32),
          pltpu.VMEM((BPW, DC), jnp.float32),
          pltpu.SemaphoreType.DMA,
          pltpu.SemaphoreType.DMA,
      ],
  )
  def k(x_h, g0_h, g1_h, g2_h, g3_h, coll_h, movie_h, ov_h, ecoll_h,
        out_m, out_cl, out_ov, out_g0, out_g1, out_g2, out_g3,
        idx_v, g0_v, g1_v, g2_v, g3_v, cidx_v, movie_v, ov_v, cl_v,
        sem_ids, sem):
    wid = lax.axis_index("s") * NC + lax.axis_index("c")
    base = wid * BPW
    pltpu.sync_copy(x_h.at[pl.ds(base, BPW)], idx_v)
    cp_g0 = pltpu.async_copy(g0_h.at[idx_v], g0_v, sem_ids)
    cp_g1 = pltpu.async_copy(g1_h.at[idx_v], g1_v, sem_ids)
    cp_g2 = pltpu.async_copy(g2_h.at[idx_v], g2_v, sem_ids)
    cp_g3 = pltpu.async_copy(g3_h.at[idx_v], g3_v, sem_ids)
    cp_ci = pltpu.async_copy(coll_h.at[idx_v], cidx_v, sem_ids)
    cp_movie = pltpu.async_copy(movie_h.at[idx_v], movie_v, sem)
    cp_ov = pltpu.async_copy(ov_h.at[idx_v], ov_v, sem)

    cp_g0.wait()
    cp_g1.wait()
    cp_g2.wait()
    cp_g3.wait()
    cp_ci.wait()

    cp_cl = pltpu.async_copy(ecoll_h.at[cidx_v], cl_v, sem)

    pltpu.sync_copy(g0_v, out_g0.at[pl.ds(base, BPW)])
    pltpu.sync_copy(g1_v, out_g1.at[pl.ds(base, BPW)])
    pltpu.sync_copy(g2_v, out_g2.at[pl.ds(base, BPW)])
    pltpu.sync_copy(g3_v, out_g3.at[pl.ds(base, BPW)])

    cp_movie.wait()
    pltpu.sync_copy(movie_v, out_m.at[pl.ds(base, BPW)])
    cp_ov.wait()
    pltpu.sync_copy(ov_v, out_ov.at[pl.ds(base, BPW)])
    cp_cl.wait()
    pltpu.sync_copy(cl_v, out_cl.at[pl.ds(base, BPW)])

  return k(x, g_cols[0], g_cols[1], g_cols[2], g_cols[3], coll,
           movie_table, ov_emb, emb_collection)


_TC_BLK = 512


def _tc_body(m_ref, cl_ref, ov_ref, g0_ref, g1_ref, g2_ref, g3_ref,
             w1m_ref, w1g_ref, w1c_ref, w1ov_ref, b1_ref,
             eg_ref, w2_ref, b2_ref, out_ref):
  dot = functools.partial(jnp.dot, preferred_element_type=jnp.float32)
  eg1 = dot(eg_ref[...], w1g_ref[...])
  vocab = lax.broadcasted_iota(jnp.int32, (_TC_BLK, GENRE_VOCAB), 1)
  oh = (jnp.where(g0_ref[...] == vocab, 0.25, 0.0)
        + jnp.where(g1_ref[...] == vocab, 0.25, 0.0)
        + jnp.where(g2_ref[...] == vocab, 0.25, 0.0)
        + jnp.where(g3_ref[...] == vocab, 0.25, 0.0))
  h = (dot(m_ref[...], w1m_ref[...])
       + dot(oh, eg1)
       + dot(cl_ref[...], w1c_ref[...])
       + dot(ov_ref[...], w1ov_ref[...])
       + b1_ref[...])
  h = jnp.maximum(h, 0.0)
  out_ref[...] = dot(h, w2_ref[...]) + b2_ref[...]


def _tc_dense(m, cl, ov, g0, g1, g2, g3, w1m, w1g, w1c, w1ov, b1, eg, w2, b2):
  grid = (B // _TC_BLK,)
  bs = pl.BlockSpec
  return pl.pallas_call(
      _tc_body,
      grid=grid,
      in_specs=[
          bs((_TC_BLK, D_MOVIE), lambda i: (i, 0)),
          bs((_TC_BLK, DC), lambda i: (i, 0)),
          bs((_TC_BLK, D_OV), lambda i: (i, 0)),
          bs((_TC_BLK, 1), lambda i: (i, 0)),
          bs((_TC_BLK, 1), lambda i: (i, 0)),
          bs((_TC_BLK, 1), lambda i: (i, 0)),
          bs((_TC_BLK, 1), lambda i: (i, 0)),
          bs((D_MOVIE, HIDDEN), lambda i: (0, 0)),
          bs((GENRE_VOCAB, HIDDEN), lambda i: (0, 0)),
          bs((DC, HIDDEN), lambda i: (0, 0)),
          bs((D_OV, HIDDEN), lambda i: (0, 0)),
          bs((1, HIDDEN), lambda i: (0, 0)),
          bs((GENRE_VOCAB, DG), lambda i: (0, 0)),
          bs((HIDDEN, RANK), lambda i: (0, 0)),
          bs((1, RANK), lambda i: (0, 0)),
      ],
      out_specs=bs((_TC_BLK, RANK), lambda i: (i, 0)),
      out_shape=jax.ShapeDtypeStruct((B, RANK), jnp.float32),
  )(m, cl, ov, g0, g1, g2, g3, w1m, w1g, w1c, w1ov, b1, eg, w2, b2)


def kernel(x, genres, collection, ov_emb, movie_table, emb_genre,
           emb_collection, W1, b1, W2, b2):
  x = x.astype(jnp.int32)
  genres = genres.astype(jnp.int32)
  g_cols = [genres[:, j] for j in range(G)]
  coll = collection.astype(jnp.int32)

  m, cl, ov, g0, g1, g2, g3 = _sc_gather(
      x, g_cols, coll, movie_table, ov_emb, emb_collection)

  w1m = W1[:D_MOVIE]
  w1g = W1[D_MOVIE:D_MOVIE + DG]
  w1c = W1[D_MOVIE + DG:D_MOVIE + DG + DC]
  w1ov = W1[D_MOVIE + DG + DC:]
  return _tc_dense(m, cl, ov,
                   g0[:, None], g1[:, None], g2[:, None], g3[:, None],
                   w1m, w1g, w1c, w1ov,
                   b1[None, :], emb_genre, W2, b2[None, :])


# speedup vs baseline: 1.5531x; 1.5342x over previous
import functools

import jax
import jax.numpy as jnp
from jax import lax
from jax.experimental import pallas as pl
from jax.experimental.pallas import tpu as pltpu
from jax.experimental.pallas import tpu_sc as plsc

B = 4096
D_MOVIE = 64
D_OV = 384
G = 4
GENRE_VOCAB = 32
DG = 32
DC = 32
HIDDEN = 64
RANK = 64
D_IN = D_MOVIE + DG + DC + D_OV

NC = 2
NS = 16
NW = NC * NS
BPW = B // NW


def _mesh():
  return plsc.VectorSubcoreMesh(
      core_axis_name="c", subcore_axis_name="s", num_cores=NC, num_subcores=NS)


def _sc_gather_a(x, genres_flat, ov_emb):
  @functools.partial(
      pl.kernel,
      out_type=(
          jax.ShapeDtypeStruct((B, D_OV), jnp.float32),
          jax.ShapeDtypeStruct((B,), jnp.int32),
          jax.ShapeDtypeStruct((B,), jnp.int32),
          jax.ShapeDtypeStruct((B,), jnp.int32),
          jax.ShapeDtypeStruct((B,), jnp.int32),
      ),
      mesh=_mesh(),
      scratch_types=[
          pltpu.VMEM((BPW,), jnp.int32),
          pltpu.VMEM((BPW,), jnp.int32),
          pltpu.VMEM((BPW,), jnp.int32),
          pltpu.VMEM((BPW,), jnp.int32),
          pltpu.VMEM((BPW,), jnp.int32),
          pltpu.VMEM((BPW,), jnp.int32),
          pltpu.VMEM((BPW,), jnp.int32),
          pltpu.VMEM((BPW,), jnp.int32),
          pltpu.VMEM((BPW,), jnp.int32),
          pltpu.VMEM((BPW, D_OV), jnp.float32),
          pltpu.SemaphoreType.DMA,
          pltpu.SemaphoreType.DMA,
          pltpu.SemaphoreType.DMA,
          pltpu.SemaphoreType.DMA,
          pltpu.SemaphoreType.DMA,
      ],
  )
  def k(x_h, gflat_h, ov_h,
        out_ov, out_g0, out_g1, out_g2, out_g3,
        idx_v, i0_v, i1_v, i2_v, i3_v, g0_v, g1_v, g2_v, g3_v, ov_v,
        sem_ov, sem0, sem1, sem2, sem3):
    wid = lax.axis_index("s") * NC + lax.axis_index("c")
    base = wid * BPW
    pltpu.sync_copy(x_h.at[pl.ds(base, BPW)], idx_v)
    cp_ov = pltpu.async_copy(ov_h.at[idx_v], ov_v, sem_ov)

    def idx_body(i, carry):
      v4 = idx_v[pl.ds(i * 16, 16)] * G
      i0_v[pl.ds(i * 16, 16)] = v4
      i1_v[pl.ds(i * 16, 16)] = v4 + 1
      i2_v[pl.ds(i * 16, 16)] = v4 + 2
      i3_v[pl.ds(i * 16, 16)] = v4 + 3
      return carry

    lax.fori_loop(0, BPW // 16, idx_body, 0)

    cp_g0 = pltpu.async_copy(gflat_h.at[i0_v], g0_v, sem0)
    cp_g1 = pltpu.async_copy(gflat_h.at[i1_v], g1_v, sem1)
    cp_g2 = pltpu.async_copy(gflat_h.at[i2_v], g2_v, sem2)
    cp_g3 = pltpu.async_copy(gflat_h.at[i3_v], g3_v, sem3)

    cp_g0.wait()
    pltpu.sync_copy(g0_v, out_g0.at[pl.ds(base, BPW)])
    cp_g1.wait()
    pltpu.sync_copy(g1_v, out_g1.at[pl.ds(base, BPW)])
    cp_g2.wait()
    pltpu.sync_copy(g2_v, out_g2.at[pl.ds(base, BPW)])
    cp_g3.wait()
    pltpu.sync_copy(g3_v, out_g3.at[pl.ds(base, BPW)])

    cp_ov.wait()
    pltpu.sync_copy(ov_v, out_ov.at[pl.ds(base, BPW)])

  return k(x, genres_flat, ov_emb)


def _sc_gather_b(x, coll, movie_table, emb_collection):
  @functools.partial(
      pl.kernel,
      out_type=(
          jax.ShapeDtypeStruct((B, D_MOVIE), jnp.float32),
          jax.ShapeDtypeStruct((B, DC), jnp.float32),
      ),
      mesh=_mesh(),
      compiler_params=pltpu.CompilerParams(use_tc_tiling_on_sc=False),
      scratch_types=[
          pltpu.VMEM((BPW,), jnp.int32),
          pltpu.VMEM((BPW,), jnp.int32),
          pltpu.VMEM((BPW, D_MOVIE), jnp.float32),
          pltpu.VMEM((BPW, DC), jnp.float32),
          pltpu.SemaphoreType.DMA,
          pltpu.SemaphoreType.DMA,
          pltpu.SemaphoreType.DMA,
      ],
  )
  def k(x_h, coll_h, movie_h, ecoll_h,
        out_m, out_cl,
        idx_v, cidx_v, movie_v, cl_v, sem_ids, sem_m, sem_cl):
    wid = lax.axis_index("s") * NC + lax.axis_index("c")
    base = wid * BPW
    pltpu.sync_copy(x_h.at[pl.ds(base, BPW)], idx_v)
    cp_ci = pltpu.async_copy(coll_h.at[idx_v], cidx_v, sem_ids)
    cp_movie = pltpu.async_copy(movie_h.at[idx_v], movie_v, sem_m)
    cp_ci.wait()
    cp_cl = pltpu.async_copy(ecoll_h.at[cidx_v], cl_v, sem_cl)
    cp_movie.wait()
    pltpu.sync_copy(movie_v, out_m.at[pl.ds(base, BPW)])
    cp_cl.wait()
    pltpu.sync_copy(cl_v, out_cl.at[pl.ds(base, BPW)])

  return k(x, coll, movie_table, emb_collection)


_TC_BLK = 512


def _tc_body(m_ref, cl_ref, ov_ref, g0_ref, g1_ref, g2_ref, g3_ref,
             w1_ref, b1_ref, eg_ref, w2_ref, b2_ref, out_ref):
  dot = functools.partial(jnp.dot, preferred_element_type=jnp.float32)
  w1g = w1_ref[D_MOVIE:D_MOVIE + DG, :]
  eg1 = dot(eg_ref[...], w1g)
  vocab = lax.broadcasted_iota(jnp.int32, (_TC_BLK, GENRE_VOCAB), 1)
  oh = (jnp.where(g0_ref[...] == vocab, 0.25, 0.0)
        + jnp.where(g1_ref[...] == vocab, 0.25, 0.0)
        + jnp.where(g2_ref[...] == vocab, 0.25, 0.0)
        + jnp.where(g3_ref[...] == vocab, 0.25, 0.0))
  h = (dot(m_ref[...], w1_ref[:D_MOVIE, :])
       + dot(oh, eg1)
       + dot(cl_ref[...], w1_ref[D_MOVIE + DG:D_MOVIE + DG + DC, :])
       + dot(ov_ref[...], w1_ref[D_MOVIE + DG + DC:, :])
       + b1_ref[...])
  h = jnp.maximum(h, 0.0)
  out_ref[...] = dot(h, w2_ref[...]) + b2_ref[...]


def _tc_dense(m, cl, ov, g0, g1, g2, g3, w1, b1, eg, w2, b2):
  grid = (B // _TC_BLK,)
  bs = pl.BlockSpec
  return pl.pallas_call(
      _tc_body,
      grid=grid,
      in_specs=[
          bs((_TC_BLK, D_MOVIE), lambda i: (i, 0)),
          bs((_TC_BLK, DC), lambda i: (i, 0)),
          bs((_TC_BLK, D_OV), lambda i: (i, 0)),
          bs((_TC_BLK, 1), lambda i: (i, 0)),
          bs((_TC_BLK, 1), lambda i: (i, 0)),
          bs((_TC_BLK, 1), lambda i: (i, 0)),
          bs((_TC_BLK, 1), lambda i: (i, 0)),
          bs((D_IN, HIDDEN), lambda i: (0, 0)),
          bs((1, HIDDEN), lambda i: (0, 0)),
          bs((GENRE_VOCAB, DG), lambda i: (0, 0)),
          bs((HIDDEN, RANK), lambda i: (0, 0)),
          bs((1, RANK), lambda i: (0, 0)),
      ],
      out_specs=bs((_TC_BLK, RANK), lambda i: (i, 0)),
      out_shape=jax.ShapeDtypeStruct((B, RANK), jnp.float32),
  )(m, cl, ov, g0, g1, g2, g3, w1, b1, eg, w2, b2)


def kernel(x, genres, collection, ov_emb, movie_table, emb_genre,
           emb_collection, W1, b1, W2, b2):
  x = x.astype(jnp.int32)
  genres_flat = genres.astype(jnp.int32).reshape(-1)
  coll = collection.astype(jnp.int32)

  ov, g0, g1, g2, g3 = _sc_gather_a(x, genres_flat, ov_emb)
  m, cl = _sc_gather_b(x, coll, movie_table, emb_collection)
  return _tc_dense(m, cl, ov,
                   g0[:, None], g1[:, None], g2[:, None], g3[:, None],
                   W1, b1[None, :], emb_genre, W2, b2[None, :])
